# baseline (speedup 1.0000x reference)
import jax
import jax.numpy as jnp
from jax.experimental import pallas as pl
from jax.experimental.pallas import tpu as pltpu

_BATCH_BLK = 32
_PE_ROWS = 20


def _pe_add_kernel(x_ref, pe_ref, inds_ref, o_ref):
    inds = inds_ref[...]
    seq = inds.shape[0]
    iota = jax.lax.broadcasted_iota(jnp.int32, (seq, _PE_ROWS), 1)
    onehot = (inds == iota).astype(jnp.float32)
    fpe = jnp.dot(onehot, pe_ref[...], preferred_element_type=jnp.float32)
    o_ref[...] = x_ref[...] + fpe[None, :, :]


def kernel(x, pe, inds):
    batch, seq, d_model = x.shape
    inds2d = inds.astype(jnp.int32).reshape(seq, 1)
    grid = (batch // _BATCH_BLK,)
    return pl.pallas_call(
        _pe_add_kernel,
        grid=grid,
        in_specs=[
            pl.BlockSpec((_BATCH_BLK, seq, d_model), lambda i: (i, 0, 0)),
            pl.BlockSpec((_PE_ROWS, d_model), lambda i: (0, 0)),
            pl.BlockSpec((seq, 1), lambda i: (0, 0)),
        ],
        out_specs=pl.BlockSpec((_BATCH_BLK, seq, d_model), lambda i: (i, 0, 0)),
        out_shape=jax.ShapeDtypeStruct((batch, seq, d_model), jnp.float32),
        compiler_params=pltpu.CompilerParams(
            dimension_semantics=("arbitrary",),
        ),
    )(x, pe, inds2d)
